# baseline (speedup 1.0000x reference)
import functools

import jax
import jax.numpy as jnp
from jax import lax
from jax.experimental import pallas as pl
from jax.experimental.pallas import tpu as pltpu
from jax.experimental.pallas import tpu_sc as plsc

B, L, V, D = 16384, 200, 128, 8
N = B * L
NC, NS = 2, 16
NW = NC * NS
PER_W = N // NW
C = 4096
NCHUNK = PER_W // C


def _fuse_body(wte_ref, w_ref, b_ref, out_ref):
    out_ref[...] = lax.dot_general(
        wte_ref[...], w_ref[...],
        dimension_numbers=(((1,), (1,)), ((), ())),
        preferred_element_type=jnp.float32,
    ) + b_ref[...]


_fuse_table = pl.pallas_call(
    _fuse_body,
    out_shape=jax.ShapeDtypeStruct((V, D), jnp.float32),
)


K = 128
NJ = C // K


def _sc_body(ft_hbm, idx_hbm, out_hbm, idx_v, rows_v, sem_in, sem_g, sem_out):
    wid = lax.axis_index("s") * NC + lax.axis_index("c")
    rbase = wid * (PER_W // K)

    pltpu.async_copy(idx_hbm.at[pl.ds(rbase, NJ), :], idx_v.at[0], sem_in)

    @pl.loop(0, NCHUNK)
    def _chunk(c):
        slot = c % 2

        pltpu.make_async_copy(
            idx_hbm.at[pl.ds(rbase + c * NJ, NJ), :],
            idx_v.at[slot], sem_in).wait()

        @pl.when(c + 1 < NCHUNK)
        def _():
            pltpu.async_copy(
                idx_hbm.at[pl.ds(rbase + (c + 1) * NJ, NJ), :],
                idx_v.at[1 - slot], sem_in)

        @pl.when(c >= 2)
        def _():
            pltpu.make_async_copy(
                rows_v.at[slot],
                out_hbm.at[pl.ds((rbase + (c - 2) * NJ) * K, C), :],
                sem_out).wait()

        @pl.loop(0, NJ, unroll=4)
        def _fire(j):
            pltpu.async_copy(
                ft_hbm.at[idx_v.at[slot, j]],
                rows_v.at[slot, pl.ds(j * K, K), :], sem_g)

        pltpu.make_async_copy(
            out_hbm.at[pl.ds((rbase + c * NJ) * K, C), :],
            rows_v.at[slot], sem_g).wait()

        pltpu.async_copy(
            rows_v.at[slot],
            out_hbm.at[pl.ds((rbase + c * NJ) * K, C), :], sem_out)

    for t in (NCHUNK - 2, NCHUNK - 1):
        pltpu.make_async_copy(
            rows_v.at[t % 2],
            out_hbm.at[pl.ds((rbase + t * NJ) * K, C), :], sem_out).wait()


_sc_gather = pl.kernel(
    _sc_body,
    out_type=jax.ShapeDtypeStruct((N, D), jnp.float32),
    mesh=plsc.VectorSubcoreMesh(
        core_axis_name="c", subcore_axis_name="s",
        num_cores=NC, num_subcores=NS),
    compiler_params=pltpu.CompilerParams(
        needs_layout_passes=False, use_tc_tiling_on_sc=False),
    scratch_types=[
        pltpu.VMEM((2, NJ, K), jnp.int32),
        pltpu.VMEM((2, C, D), jnp.float32),
        pltpu.SemaphoreType.DMA,
        pltpu.SemaphoreType.DMA,
        pltpu.SemaphoreType.DMA,
    ],
)


@jax.jit
def kernel(x, wte, W, b):
    ft = _fuse_table(wte, W, b.reshape(1, D))
    out = _sc_gather(ft, x.reshape(N // K, K).astype(jnp.int32))
    return out.reshape(B, L, D)
